# baseline (speedup 1.0000x reference)
import functools

import jax
import jax.numpy as jnp
from jax import lax
from jax.experimental import pallas as pl
from jax.experimental.pallas import tpu as pltpu
from jax.experimental.pallas import tpu_sc as plsc

B = 32; S_IMG = 256; C_IN = 768; D = 768; H = 12; DH = 64; NL = 12
F = 3072; E = 8; NQ = 32; OUT = 4096
T = B * NQ
CAP = 192
NSLOT = E * CAP
NW = 32
TPW = T // NW
XE_ROWS = NSLOT + NW

_INTERP = False


def _stats(z):
    return (jnp.mean(z, -1, keepdims=True), jnp.var(z, -1, keepdims=True))


def _norm(x, m, v, g, b):
    return (x - m) / jnp.sqrt(v + 1e-12) * g + b


def _dot(a, b):
    return jnp.dot(a.astype(jnp.bfloat16), b.astype(jnp.bfloat16),
                   preferred_element_type=jnp.float32)


def _dot_t(a, b):
    return lax.dot_general(a.astype(jnp.bfloat16), b.astype(jnp.bfloat16),
                           (((1,), (1,)), ((), ())),
                           preferred_element_type=jnp.float32)


def _mm_bias_body(x_ref, w_ref, b_ref, o_ref):
    o_ref[...] = _dot(x_ref[...], w_ref[...]) + b_ref[...]


def _in_proj(img2, w, b2):
    return pl.pallas_call(
        _mm_bias_body,
        grid=(8,),
        in_specs=[
            pl.BlockSpec((1024, C_IN), lambda i: (i, 0)),
            pl.BlockSpec((C_IN, D), lambda i: (0, 0)),
            pl.BlockSpec((1, D), lambda i: (0, 0)),
        ],
        out_specs=pl.BlockSpec((1024, D), lambda i: (i, 0)),
        out_shape=jax.ShapeDtypeStruct((B * S_IMG, D), jnp.float32),
        interpret=_INTERP,
    )(img2, w, b2)


def _kv_body(x_ref, wk_ref, bk_ref, wv_ref, bv_ref, k_ref, v_ref):
    x = x_ref[...]
    k_ref[...] = _dot(x, wk_ref[...]) + bk_ref[...]
    v_ref[...] = _dot(x, wv_ref[...]) + bv_ref[...]


def _ca_kv(imgp, wk, bk2, wv, bv2):
    return pl.pallas_call(
        _kv_body,
        grid=(8,),
        in_specs=[
            pl.BlockSpec((1024, D), lambda i: (i, 0)),
            pl.BlockSpec((D, D), lambda i: (0, 0)),
            pl.BlockSpec((1, D), lambda i: (0, 0)),
            pl.BlockSpec((D, D), lambda i: (0, 0)),
            pl.BlockSpec((1, D), lambda i: (0, 0)),
        ],
        out_specs=[
            pl.BlockSpec((1024, D), lambda i: (i, 0)),
            pl.BlockSpec((1024, D), lambda i: (i, 0)),
        ],
        out_shape=[
            jax.ShapeDtypeStruct((B * S_IMG, D), jnp.float32),
            jax.ShapeDtypeStruct((B * S_IMG, D), jnp.float32),
        ],
        interpret=_INTERP,
    )(imgp, wk, bk2, wv, bv2)


_FULL = pl.BlockSpec((T, D), lambda: (0, 0))
_WSP = pl.BlockSpec((D, D), lambda: (0, 0))
_BSP = pl.BlockSpec((1, D), lambda: (0, 0))
_CSP = pl.BlockSpec((T, 1), lambda: (0, 0))


def _sa_body(z_ref, m_ref, v_ref, g_ref, b_ref, wq_ref, bq_ref, wk_ref,
             bk_ref, wv_ref, bv_ref, wo_ref, bo_ref, o_ref):
    x = _norm(z_ref[...], m_ref[...], v_ref[...], g_ref[...], b_ref[...])
    q = _dot(x, wq_ref[...]) + bq_ref[...]
    k = _dot(x, wk_ref[...]) + bk_ref[...]
    v = _dot(x, wv_ref[...]) + bv_ref[...]
    r = lax.broadcasted_iota(jnp.int32, (T, T), 0) // NQ
    c = lax.broadcasted_iota(jnp.int32, (T, T), 1) // NQ
    mask = jnp.where(r == c, 0.0, -1e30).astype(jnp.float32)
    ctxs = []
    for hh in range(H):
        sl = slice(hh * DH, (hh + 1) * DH)
        s = _dot_t(q[:, sl], k[:, sl]) / 8.0 + mask
        mx = jnp.max(s, -1, keepdims=True)
        e = jnp.exp(s - mx)
        p = e / jnp.sum(e, -1, keepdims=True)
        ctxs.append(_dot(p, v[:, sl]))
    ctx = jnp.concatenate(ctxs, axis=1)
    o_ref[...] = x + _dot(ctx, wo_ref[...]) + bo_ref[...]


def _sa_layer(z, m, v, g, b, wq, bq, wk, bk, wv, bv, wo, bo):
    return pl.pallas_call(
        _sa_body,
        in_specs=[_FULL, _CSP, _CSP, _BSP, _BSP, _WSP, _BSP, _WSP, _BSP,
                  _WSP, _BSP, _WSP, _BSP],
        out_specs=_FULL,
        out_shape=jax.ShapeDtypeStruct((T, D), jnp.float32),
        interpret=_INTERP,
    )(z, m, v, g, b, wq, bq, wk, bk, wv, bv, wo, bo)


def _ca_q_body(z_ref, m_ref, v_ref, g_ref, b_ref, w_ref, bq_ref, o_ref):
    h = _norm(z_ref[...], m_ref[...], v_ref[...], g_ref[...], b_ref[...])
    o_ref[...] = _dot(h, w_ref[...]) + bq_ref[...]


def _ca_q(z, m, v, g, b, wq, bq2):
    return pl.pallas_call(
        _ca_q_body,
        in_specs=[_FULL, _CSP, _CSP, _BSP, _BSP, _WSP, _BSP],
        out_specs=_FULL,
        out_shape=jax.ShapeDtypeStruct((T, D), jnp.float32),
        interpret=_INTERP,
    )(z, m, v, g, b, wq, bq2)


def _ca_attn_body(q_ref, k_ref, v_ref, o_ref):
    q = q_ref[...]
    k = k_ref[...]
    v = v_ref[...]
    ctxs = []
    for hh in range(H):
        sl = slice(hh * DH, (hh + 1) * DH)
        s = _dot_t(q[:, sl], k[:, sl]) / 8.0
        mx = jnp.max(s, -1, keepdims=True)
        e = jnp.exp(s - mx)
        p = e / jnp.sum(e, -1, keepdims=True)
        ctxs.append(_dot(p, v[:, sl]))
    o_ref[...] = jnp.concatenate(ctxs, axis=1)


def _ca_attn(qf, kf, vf):
    return pl.pallas_call(
        _ca_attn_body,
        grid=(B,),
        in_specs=[pl.BlockSpec((NQ, D), lambda i: (i, 0)),
                  pl.BlockSpec((S_IMG, D), lambda i: (i, 0)),
                  pl.BlockSpec((S_IMG, D), lambda i: (i, 0))],
        out_specs=pl.BlockSpec((NQ, D), lambda i: (i, 0)),
        out_shape=jax.ShapeDtypeStruct((T, D), jnp.float32),
        interpret=_INTERP,
    )(qf, kf, vf)


def _ca_out_body(ctx_ref, wo_ref, bo_ref, z_ref, m_ref, v_ref, g_ref,
                 b_ref, o_ref):
    h = _norm(z_ref[...], m_ref[...], v_ref[...], g_ref[...], b_ref[...])
    o_ref[...] = h + _dot(ctx_ref[...], wo_ref[...]) + bo_ref[...]


def _ca_out(ctx, wo, bo2, z, m, v, g, b):
    return pl.pallas_call(
        _ca_out_body,
        in_specs=[_FULL, _WSP, _BSP, _FULL, _CSP, _CSP, _BSP, _BSP],
        out_specs=_FULL,
        out_shape=jax.ShapeDtypeStruct((T, D), jnp.float32),
        interpret=_INTERP,
    )(ctx, wo, bo2, z, m, v, g, b)


def _router_body(z_ref, m_ref, v_ref, g_ref, b_ref, wr_ref, br_ref,
                 h_ref, didx_ref, cidx_ref, scale_ref):
    x = _norm(z_ref[...], m_ref[...], v_ref[...], g_ref[...], b_ref[...])
    h_ref[...] = x
    logits = _dot(x, wr_ref[...]) + br_ref[...]
    mx = jnp.max(logits, -1, keepdims=True)
    ex = jnp.exp(logits - mx)
    probs = ex / jnp.sum(ex, -1, keepdims=True)
    top_p = jnp.max(probs, -1, keepdims=True)
    lane = lax.broadcasted_iota(jnp.int32, (T, E), 1)
    top_e = jnp.min(jnp.where(probs == top_p, lane, E), -1, keepdims=True)
    onehot = (lane == top_e).astype(jnp.float32)
    rr = lax.broadcasted_iota(jnp.int32, (T, T), 0)
    cc = lax.broadcasted_iota(jnp.int32, (T, T), 1)
    tril = (cc <= rr).astype(jnp.float32)
    cum = _dot(tril, onehot)
    pos = jnp.sum(jnp.where(lane == top_e, cum, 0.0), -1, keepdims=True) - 1.0
    keep = pos < CAP
    pos_c = jnp.minimum(pos, CAP - 1)
    slot = (top_e * CAP + pos_c.astype(jnp.int32))
    wid = lax.broadcasted_iota(jnp.int32, (T, 1), 0) // TPW
    didx_ref[...] = jnp.where(keep, slot, NSLOT + wid)
    cidx_ref[...] = jnp.where(keep, slot, 0)
    scale_ref[...] = jnp.where(keep, top_p, 0.0)


def _router(z, m, v, g, b, wr, br2):
    return pl.pallas_call(
        _router_body,
        in_specs=[_FULL, _CSP, _CSP, _BSP, _BSP,
                  pl.BlockSpec((D, E), lambda: (0, 0)),
                  pl.BlockSpec((1, E), lambda: (0, 0))],
        out_specs=[_FULL, _CSP, _CSP, _CSP],
        out_shape=[jax.ShapeDtypeStruct((T, D), jnp.float32),
                   jax.ShapeDtypeStruct((T, 1), jnp.int32),
                   jax.ShapeDtypeStruct((T, 1), jnp.int32),
                   jax.ShapeDtypeStruct((T, 1), jnp.float32)],
        interpret=_INTERP,
    )(z, m, v, g, b, wr, br2)


_SC_MESH = None


def _sc_mesh():
    global _SC_MESH
    if _SC_MESH is None:
        _SC_MESH = plsc.VectorSubcoreMesh(core_axis_name="c",
                                          subcore_axis_name="s")
    return _SC_MESH


def _dispatch(x, didx3):
    @functools.partial(
        pl.kernel, mesh=_sc_mesh(),
        out_type=jax.ShapeDtypeStruct((XE_ROWS, D), jnp.float32),
        scratch_types=[pltpu.VMEM((1, TPW), jnp.int32),
                       pltpu.VMEM((TPW, D), jnp.float32),
                       pltpu.SemaphoreType.DMA],
    )
    def k(x_hbm, idx_hbm, xe_hbm, idx_v, rows_v, sem):
        wid = lax.axis_index("s") * 2 + lax.axis_index("c")
        base = wid * TPW
        pltpu.sync_copy(idx_hbm.at[wid], idx_v)
        pltpu.sync_copy(x_hbm.at[pl.ds(base, TPW)], rows_v)
        pltpu.async_copy(rows_v, xe_hbm.at[idx_v.at[0]], sem).wait()

    return k(x, didx3)


def _combine(ye, cidx1):
    @functools.partial(
        pl.kernel, mesh=_sc_mesh(),
        out_type=jax.ShapeDtypeStruct((T, D), jnp.float32),
        scratch_types=[pltpu.VMEM((TPW,), jnp.int32),
                       pltpu.VMEM((TPW, D), jnp.float32),
                       pltpu.SemaphoreType.DMA],
    )
    def k(ye_hbm, idx_hbm, y_hbm, idx_v, rows_v, sem):
        wid = lax.axis_index("s") * 2 + lax.axis_index("c")
        base = wid * TPW
        pltpu.sync_copy(idx_hbm.at[pl.ds(base, TPW)], idx_v)
        pltpu.async_copy(ye_hbm.at[idx_v], rows_v, sem).wait()
        pltpu.sync_copy(rows_v, y_hbm.at[pl.ds(base, TPW)])

    return k(ye, cidx1)


def _ffn_body(xe_ref, w1_ref, b1_ref, w2_ref, b2_ref, ye_ref):
    x = xe_ref[...]
    hid = jax.nn.gelu(_dot(x, w1_ref[0]) + b1_ref[0])
    ye_ref[...] = _dot(hid, w2_ref[0]) + b2_ref[0]


def _ffn(xe, w1, b1, w2, b2):
    return pl.pallas_call(
        _ffn_body,
        grid=(E,),
        in_specs=[pl.BlockSpec((CAP, D), lambda e: (e, 0)),
                  pl.BlockSpec((1, D, F), lambda e: (e, 0, 0)),
                  pl.BlockSpec((1, 1, F), lambda e: (e, 0, 0)),
                  pl.BlockSpec((1, F, D), lambda e: (e, 0, 0)),
                  pl.BlockSpec((1, 1, D), lambda e: (e, 0, 0))],
        out_specs=pl.BlockSpec((CAP, D), lambda e: (e, 0)),
        out_shape=jax.ShapeDtypeStruct((NSLOT, D), jnp.float32),
        interpret=_INTERP,
    )(xe, w1, b1, w2, b2)


def _epi_body(h_ref, y_ref, s_ref, o_ref):
    o_ref[...] = h_ref[...] + y_ref[...] * s_ref[...]


def _epilogue(h, y, scale):
    return pl.pallas_call(
        _epi_body,
        in_specs=[_FULL, _FULL, _CSP],
        out_specs=_FULL,
        out_shape=jax.ShapeDtypeStruct((T, D), jnp.float32),
        interpret=_INTERP,
    )(h, y, scale)


def _llm_body(z_ref, m_ref, v_ref, g_ref, b_ref, w_ref, bo_ref, o_ref):
    h = _norm(z_ref[...], m_ref[...], v_ref[...], g_ref[...], b_ref[...])
    o_ref[...] = _dot(h, w_ref[...]) + bo_ref[...]


def _llm(z, m, v, g, b, w, b2):
    return pl.pallas_call(
        _llm_body,
        grid=(4,),
        in_specs=[pl.BlockSpec((T, D), lambda j: (0, 0)),
                  pl.BlockSpec((T, 1), lambda j: (0, 0)),
                  pl.BlockSpec((T, 1), lambda j: (0, 0)),
                  pl.BlockSpec((1, D), lambda j: (0, 0)),
                  pl.BlockSpec((1, D), lambda j: (0, 0)),
                  pl.BlockSpec((D, 1024), lambda j: (0, j)),
                  pl.BlockSpec((1, 1024), lambda j: (0, j))],
        out_specs=pl.BlockSpec((T, 1024), lambda j: (0, j)),
        out_shape=jax.ShapeDtypeStruct((T, OUT), jnp.float32),
        interpret=_INTERP,
    )(z, m, v, g, b, w, b2)


def kernel(image_features, in_proj_w, in_proj_b, query_tokens, ln_emb_g,
           ln_emb_b, sa_wq, sa_wk, sa_wv, sa_wo, sa_bq, sa_bk, sa_bv, sa_bo,
           sa_ln_g, sa_ln_b, ca_wq, ca_wk, ca_wv, ca_wo, ca_bq, ca_bk,
           ca_bv, ca_bo, ca_ln_g, ca_ln_b, router_w, router_b, exp_w1,
           exp_b1, exp_w2, exp_b2, ffn_ln_g, ffn_ln_b, llm_w, llm_b):
    r2 = lambda a: a.reshape(1, -1)
    img2 = image_features.reshape(B * S_IMG, C_IN)
    imgp = _in_proj(img2, in_proj_w, r2(in_proj_b))
    z = jnp.broadcast_to(query_tokens, (B, NQ, D)).reshape(T, D)
    m, v = _stats(z)
    g, b = r2(ln_emb_g), r2(ln_emb_b)
    for l in range(NL):
        z = _sa_layer(z, m, v, g, b, sa_wq[l], r2(sa_bq[l]), sa_wk[l],
                      r2(sa_bk[l]), sa_wv[l], r2(sa_bv[l]), sa_wo[l],
                      r2(sa_bo[l]))
        m, v = _stats(z)
        g, b = r2(sa_ln_g[l]), r2(sa_ln_b[l])
        if l % 2 == 0:
            qf = _ca_q(z, m, v, g, b, ca_wq[l], r2(ca_bq[l]))
            kf, vf = _ca_kv(imgp, ca_wk[l], r2(ca_bk[l]),
                            ca_wv[l], r2(ca_bv[l]))
            ctx = _ca_attn(qf, kf, vf)
            z = _ca_out(ctx, ca_wo[l], r2(ca_bo[l]), z, m, v, g, b)
            m, v = _stats(z)
            g, b = r2(ca_ln_g[l]), r2(ca_ln_b[l])
        h, didx, cidx, scale = _router(z, m, v, g, b, router_w[l],
                                       r2(router_b[l]))
        xe = _dispatch(h, didx.reshape(NW, 1, TPW))
        ye = _ffn(xe, exp_w1[l], exp_b1[l].reshape(E, 1, F),
                  exp_w2[l], exp_b2[l].reshape(E, 1, D))
        y = _combine(ye, cidx.reshape(T))
        z = _epilogue(h, y, scale)
        m, v = _stats(z)
        g, b = r2(ffn_ln_g[l]), r2(ffn_ln_b[l])
    out = _llm(z, m, v, g, b, llm_w, r2(llm_b))
    return out.reshape(B, NQ, OUT)
